# baseline (speedup 1.0000x reference)
import functools

import jax
import jax.numpy as jnp
from jax import lax
from jax.experimental import pallas as pl
from jax.experimental.pallas import tpu as pltpu
from jax.experimental.pallas import tpu_sc as plsc

_VOCAB = 100000
_EMB = 128
_B = 4096
_L = 50
_NV = _EMB // 16
_NC = 2
_NS = 16
_NW = _NC * _NS
_SEQ_PER_W = _B // _NW
_GTOK = 100
_NG = _SEQ_PER_W * _L // _GTOK
_K = 4
_NB = _NG // _K
_BLK_SEQ = _K * _GTOK // _L
_TAIL_BLK = 512


@functools.partial(
    pl.kernel,
    out_type=jax.ShapeDtypeStruct((_B * _EMB,), jnp.float32),
    mesh=plsc.VectorSubcoreMesh(core_axis_name="c", subcore_axis_name="s",
                                num_cores=_NC, num_subcores=_NS),
    scratch_types=[
        pltpu.VMEM((_NG, _GTOK), jnp.int32),
        pltpu.VMEM((_K, _GTOK, _EMB), jnp.float32),
        pltpu.VMEM((_K, _GTOK, _EMB), jnp.float32),
        pltpu.VMEM((_SEQ_PER_W * _EMB,), jnp.float32),
        pltpu.SemaphoreType.DMA,
        pltpu.SemaphoreType.DMA,
    ],
    compiler_params=pltpu.CompilerParams(use_tc_tiling_on_sc=False),
)
def _sc_pool(seq_hbm, emb_hbm, out_hbm, idx_v, rows_a, rows_b, out_v,
             sem_a, sem_b):
    wid = lax.axis_index("s") * _NC + lax.axis_index("c")
    pltpu.sync_copy(seq_hbm.at[pl.ds(wid * _NG, _NG)], idx_v)

    def fire(blk, rows, sem):
        for j in range(_K):
            pltpu.async_copy(emb_hbm.at[idx_v.at[blk * _K + j]],
                             rows.at[j], sem)

    def drain(blk, rows, sem):
        for j in range(_K):
            pltpu.make_async_copy(emb_hbm.at[idx_v.at[blk * _K + j]],
                                  rows.at[j], sem).wait()

    def reduce_block(blk, rows):
        def seq_body(s, carry):
            g = s // 2
            half = (s % 2) * _L

            def add_body(t5, accs):
                new = []
                for v in range(_NV):
                    a = accs[v]
                    for u in range(5):
                        a = a + rows[g, half + t5 * 5 + u,
                                     pl.ds(v * 16, 16)]
                    new.append(a)
                return tuple(new)

            accs = lax.fori_loop(
                0, _L // 5, add_body,
                tuple(jnp.zeros((16,), jnp.float32) for _ in range(_NV)))
            row = (blk * _BLK_SEQ + s) * _EMB
            for v in range(_NV):
                out_v[pl.ds(row + v * 16, 16)] = accs[v]
            return carry

        lax.fori_loop(0, _BLK_SEQ, seq_body, 0)

    fire(0, rows_a, sem_a)

    def pipe_body(i, carry):
        b0 = 2 * i
        b1 = 2 * i + 1
        fire(b1, rows_b, sem_b)
        drain(b0, rows_a, sem_a)
        reduce_block(b0, rows_a)

        @pl.when(b1 + 1 < _NB)
        def _():
            fire(b1 + 1, rows_a, sem_a)

        drain(b1, rows_b, sem_b)
        reduce_block(b1, rows_b)
        return carry

    lax.fori_loop(0, _NB // 2, pipe_body, 0)
    pltpu.sync_copy(out_v,
                    out_hbm.at[pl.ds(wid * _SEQ_PER_W * _EMB,
                                     _SEQ_PER_W * _EMB)])


def _tail_body(s_ref, w_ref, b_ref, out_ref):
    s = s_ref[...].reshape(_TAIL_BLK, _EMB)
    logits = jnp.dot(s, w_ref[...],
                     preferred_element_type=jnp.float32) + b_ref[...]
    out_ref[...] = jax.nn.softmax(logits, axis=-1)


def _dense_tail(sums_flat, wt, b2):
    return pl.pallas_call(
        _tail_body,
        grid=(_B // _TAIL_BLK,),
        in_specs=[
            pl.BlockSpec((_TAIL_BLK * _EMB,), lambda i: (i,)),
            pl.BlockSpec((_EMB, 2), lambda i: (0, 0)),
            pl.BlockSpec((1, 2), lambda i: (0, 0)),
        ],
        out_specs=pl.BlockSpec((_TAIL_BLK, 2), lambda i: (i, 0)),
        out_shape=jax.ShapeDtypeStruct((_B, 2), jnp.float32),
    )(sums_flat, wt, b2)


def kernel(seq, emb, W, b):
    seq2 = seq.astype(jnp.int32).reshape(_NW * _NG, _GTOK)
    sums_flat = _sc_pool(seq2, emb)
    wt = W.T * (1.0 / _L)
    return _dense_tail(sums_flat, wt, b.reshape(1, 2))


# speedup vs baseline: 1.0440x; 1.0440x over previous
import functools

import jax
import jax.numpy as jnp
from jax import lax
from jax.experimental import pallas as pl
from jax.experimental.pallas import tpu as pltpu
from jax.experimental.pallas import tpu_sc as plsc

_VOCAB = 100000
_EMB = 128
_B = 4096
_L = 50
_NV = _EMB // 16
_NC = 2
_NS = 16
_NW = _NC * _NS
_SEQ_PER_W = _B // _NW
_GTOK = 100
_NG = _SEQ_PER_W * _L // _GTOK
_K = 4
_NB = _NG // _K
_BLK_SEQ = _K * _GTOK // _L
_TAIL_BLK = 4096


@functools.partial(
    pl.kernel,
    out_type=jax.ShapeDtypeStruct((_B * _EMB,), jnp.float32),
    mesh=plsc.VectorSubcoreMesh(core_axis_name="c", subcore_axis_name="s",
                                num_cores=_NC, num_subcores=_NS),
    scratch_types=[
        pltpu.VMEM((_NG, _GTOK), jnp.int32),
        pltpu.VMEM((_K, _GTOK, _EMB), jnp.float32),
        pltpu.VMEM((_K, _GTOK, _EMB), jnp.float32),
        pltpu.VMEM((_SEQ_PER_W * _EMB,), jnp.float32),
        pltpu.SemaphoreType.DMA,
        pltpu.SemaphoreType.DMA,
    ],
    compiler_params=pltpu.CompilerParams(use_tc_tiling_on_sc=False),
)
def _sc_pool(seq_hbm, emb_hbm, out_hbm, idx_v, rows_a, rows_b, out_v,
             sem_a, sem_b):
    wid = lax.axis_index("s") * _NC + lax.axis_index("c")
    pltpu.sync_copy(seq_hbm.at[pl.ds(wid * _NG, _NG)], idx_v)

    def fire(blk, rows, sem):
        def body(j, carry):
            pltpu.async_copy(emb_hbm.at[idx_v.at[blk * _K + j]],
                             rows.at[j], sem)
            return carry

        lax.fori_loop(0, _K, body, 0)

    def drain(blk, rows, sem):
        def body(j, carry):
            pltpu.make_async_copy(emb_hbm.at[idx_v.at[blk * _K + j]],
                                  rows.at[j], sem).wait()
            return carry

        lax.fori_loop(0, _K, body, 0)

    def reduce_block(blk, rows):
        def seq_body(s, carry):
            g = s // 2
            half = (s % 2) * _L

            def add_body(t5, accs):
                new = []
                for v in range(_NV):
                    a = accs[v]
                    for u in range(5):
                        a = a + rows[g, half + t5 * 5 + u,
                                     pl.ds(v * 16, 16)]
                    new.append(a)
                return tuple(new)

            accs = lax.fori_loop(
                0, _L // 5, add_body,
                tuple(jnp.zeros((16,), jnp.float32) for _ in range(_NV)))
            row = (blk * _BLK_SEQ + s) * _EMB
            for v in range(_NV):
                out_v[pl.ds(row + v * 16, 16)] = accs[v]
            return carry

        lax.fori_loop(0, _BLK_SEQ, seq_body, 0)

    fire(0, rows_a, sem_a)

    def pipe_body(i, carry):
        b0 = 2 * i
        b1 = 2 * i + 1
        fire(b1, rows_b, sem_b)
        drain(b0, rows_a, sem_a)
        reduce_block(b0, rows_a)

        @pl.when(b1 + 1 < _NB)
        def _():
            fire(b1 + 1, rows_a, sem_a)

        drain(b1, rows_b, sem_b)
        reduce_block(b1, rows_b)
        return carry

    lax.fori_loop(0, _NB // 2, pipe_body, 0)
    pltpu.sync_copy(out_v,
                    out_hbm.at[pl.ds(wid * _SEQ_PER_W * _EMB,
                                     _SEQ_PER_W * _EMB)])


def _tail_body(s_ref, w_ref, b_ref, out_ref):
    s = s_ref[...].reshape(_TAIL_BLK, _EMB)
    logits = jnp.dot(s, w_ref[...],
                     preferred_element_type=jnp.float32) + b_ref[...]
    out_ref[...] = jax.nn.softmax(logits, axis=-1)


def _dense_tail(sums_flat, wt, b2):
    return pl.pallas_call(
        _tail_body,
        grid=(_B // _TAIL_BLK,),
        in_specs=[
            pl.BlockSpec((_TAIL_BLK * _EMB,), lambda i: (i,)),
            pl.BlockSpec((_EMB, 2), lambda i: (0, 0)),
            pl.BlockSpec((1, 2), lambda i: (0, 0)),
        ],
        out_specs=pl.BlockSpec((_TAIL_BLK, 2), lambda i: (i, 0)),
        out_shape=jax.ShapeDtypeStruct((_B, 2), jnp.float32),
    )(sums_flat, wt, b2)


def kernel(seq, emb, W, b):
    seq2 = seq.astype(jnp.int32).reshape(_NW * _NG, _GTOK)
    sums_flat = _sc_pool(seq2, emb)
    wt = W.T * (1.0 / _L)
    return _dense_tail(sums_flat, wt, b.reshape(1, 2))
